# Initial kernel scaffold; baseline (speedup 1.0000x reference)
import jax
import jax.numpy as jnp
from jax.experimental import pallas as pl


def kernel(x, w_gating):
    raise NotImplementedError("write your pallas kernel here")


# speedup vs baseline: 1.4891x; 1.4891x over previous
import functools
import math

import jax
import jax.numpy as jnp
from jax.experimental import pallas as pl
from jax.experimental.pallas import tpu as pltpu

B, T, D, E = 2, 2048, 1024, 8
THRESHOLD = 0.8
CAP_FACTOR_EVAL = 2.0
MIN_EXPERT_CAPACITY = 4
C = max(MIN_EXPERT_CAPACITY, min(T, math.ceil(T * CAP_FACTOR_EVAL / E)))

TT = 256


def _gating_kernel(x_ref, w_ref, disp_ref, comb_ref, aux_ref,
                   carry_ref, accm_ref, accp_ref, auxacc_ref):
    b = pl.program_id(0)
    t = pl.program_id(1)
    n_t = pl.num_programs(1)

    @pl.when(t == 0)
    def _():
        carry_ref[...] = jnp.zeros_like(carry_ref)
        accm_ref[...] = jnp.zeros_like(accm_ref)
        accp_ref[...] = jnp.zeros_like(accp_ref)

    @pl.when((b == 0) & (t == 0))
    def _():
        auxacc_ref[...] = jnp.zeros_like(auxacc_ref)

    xb = x_ref[0]
    gates = jnp.dot(xb, w_ref[...], preferred_element_type=jnp.float32)
    m = jnp.max(gates, axis=-1, keepdims=True)
    ex = jnp.exp(gates - m)
    probs = ex / jnp.sum(ex, axis=-1, keepdims=True)

    ge = probs[:, :, None] <= probs[:, None, :]
    s_incl = jnp.sum(jnp.where(ge, probs[:, None, :], 0.0), axis=-1)
    pmax = jnp.max(probs, axis=-1, keepdims=True)
    sel = (s_incl < THRESHOLD) | (probs >= pmax)
    maskf = sel.astype(jnp.float32)
    selp = probs * maskf
    renorm = jnp.maximum(jnp.sum(selp, axis=-1, keepdims=True), 1e-9)
    weights = selp / renorm

    row = jax.lax.broadcasted_iota(jnp.int32, (TT, TT), 0)
    col = jax.lax.broadcasted_iota(jnp.int32, (TT, TT), 1)
    ltri = (col < row).astype(jnp.float32)
    pos = jnp.dot(ltri, maskf, preferred_element_type=jnp.float32)
    pos = pos + carry_ref[0][None, :]
    carry_ref[0] = carry_ref[0] + jnp.sum(maskf, axis=0)

    accm_ref[0] = accm_ref[0] + jnp.sum(maskf, axis=0)
    accp_ref[0] = accp_ref[0] + jnp.sum(probs, axis=0)

    kc = (pos < float(C)) & sel
    kcf = kc.astype(jnp.float32)
    pos_c = jnp.minimum(pos, float(C - 1)).astype(jnp.int32)
    ciota = jax.lax.broadcasted_iota(jnp.int32, (TT, E, C), 2)
    onehot = (ciota == pos_c[:, :, None]).astype(jnp.float32)
    disp_ref[0] = onehot * kcf[:, :, None]
    comb_ref[0] = onehot * (weights * kcf)[:, :, None]

    @pl.when(t == n_t - 1)
    def _():
        auxacc_ref[...] = auxacc_ref[...] + jnp.sum(
            accm_ref[...] * accp_ref[...], axis=(0, 1), keepdims=True)

    @pl.when((b == B - 1) & (t == n_t - 1))
    def _():
        aux_ref[...] = auxacc_ref[...] * (float(E) / (float(B) * float(T) * float(T)))


@jax.jit
def kernel(x, w_gating):
    n_t = T // TT
    disp, comb, aux = pl.pallas_call(
        _gating_kernel,
        grid=(B, n_t),
        in_specs=[
            pl.BlockSpec((1, TT, D), lambda b, t: (b, t, 0)),
            pl.BlockSpec((D, E), lambda b, t: (0, 0)),
        ],
        out_specs=[
            pl.BlockSpec((1, TT, E, C), lambda b, t: (b, t, 0, 0)),
            pl.BlockSpec((1, TT, E, C), lambda b, t: (b, t, 0, 0)),
            pl.BlockSpec((1, 1), lambda b, t: (0, 0)),
        ],
        out_shape=[
            jax.ShapeDtypeStruct((B, T, E, C), jnp.float32),
            jax.ShapeDtypeStruct((B, T, E, C), jnp.float32),
            jax.ShapeDtypeStruct((1, 1), jnp.float32),
        ],
        scratch_shapes=[
            pltpu.VMEM((1, E), jnp.float32),
            pltpu.VMEM((1, E), jnp.float32),
            pltpu.VMEM((1, E), jnp.float32),
            pltpu.VMEM((1, 1), jnp.float32),
        ],
    )(x, w_gating)
    return disp, comb, aux[0, 0]


# speedup vs baseline: 1.5728x; 1.0562x over previous
import functools
import math

import jax
import jax.numpy as jnp
from jax.experimental import pallas as pl
from jax.experimental.pallas import tpu as pltpu

B, T, D, E = 2, 2048, 1024, 8
THRESHOLD = 0.8
CAP_FACTOR_EVAL = 2.0
MIN_EXPERT_CAPACITY = 4
C = max(MIN_EXPERT_CAPACITY, min(T, math.ceil(T * CAP_FACTOR_EVAL / E)))

TT = 256


def _gating_kernel(x_ref, w_ref, disp_ref, comb_ref, aux_ref,
                   carry_ref, accm_ref, accp_ref, auxacc_ref):
    b = pl.program_id(0)
    t = pl.program_id(1)
    n_t = pl.num_programs(1)

    @pl.when(t == 0)
    def _():
        carry_ref[...] = jnp.zeros_like(carry_ref)
        accm_ref[...] = jnp.zeros_like(accm_ref)
        accp_ref[...] = jnp.zeros_like(accp_ref)

    @pl.when((b == 0) & (t == 0))
    def _():
        auxacc_ref[...] = jnp.zeros_like(auxacc_ref)

    xb = x_ref[0]
    gates = jnp.dot(xb, w_ref[...], preferred_element_type=jnp.float32)
    m = jnp.max(gates, axis=-1, keepdims=True)
    ex = jnp.exp(gates - m)
    probs = ex / jnp.sum(ex, axis=-1, keepdims=True)

    ge = probs[:, :, None] <= probs[:, None, :]
    s_incl = jnp.sum(jnp.where(ge, probs[:, None, :], 0.0), axis=-1)
    pmax = jnp.max(probs, axis=-1, keepdims=True)
    sel = (s_incl < THRESHOLD) | (probs >= pmax)
    maskf = sel.astype(jnp.float32)
    selp = probs * maskf
    renorm = jnp.maximum(jnp.sum(selp, axis=-1, keepdims=True), 1e-9)
    weights = selp / renorm

    row = jax.lax.broadcasted_iota(jnp.int32, (TT, TT), 0)
    col = jax.lax.broadcasted_iota(jnp.int32, (TT, TT), 1)
    ltri = (col < row).astype(jnp.float32)
    pos = jnp.dot(ltri, maskf, preferred_element_type=jnp.float32)
    pos = pos + carry_ref[0][None, :]
    carry_ref[0] = carry_ref[0] + jnp.sum(maskf, axis=0)

    accm_ref[0] = accm_ref[0] + jnp.sum(maskf, axis=0)
    accp_ref[0] = accp_ref[0] + jnp.sum(probs, axis=0)

    kc = (pos < float(C)) & sel
    posx = jnp.where(kc, pos, float(C)).astype(jnp.int32)
    ciota = jax.lax.broadcasted_iota(jnp.int32, (TT, E, C), 2)
    onehot = (ciota == posx[:, :, None]).astype(jnp.float32)
    disp_ref[0] = onehot
    comb_ref[0] = onehot * weights[:, :, None]

    @pl.when(t == n_t - 1)
    def _():
        auxacc_ref[...] = auxacc_ref[...] + jnp.sum(
            accm_ref[...] * accp_ref[...], axis=(0, 1), keepdims=True)

    @pl.when((b == B - 1) & (t == n_t - 1))
    def _():
        aux_ref[...] = auxacc_ref[...] * (float(E) / (float(B) * float(T) * float(T)))


@jax.jit
def kernel(x, w_gating):
    n_t = T // TT
    disp, comb, aux = pl.pallas_call(
        _gating_kernel,
        grid=(B, n_t),
        in_specs=[
            pl.BlockSpec((1, TT, D), lambda b, t: (b, t, 0)),
            pl.BlockSpec((D, E), lambda b, t: (0, 0)),
        ],
        out_specs=[
            pl.BlockSpec((1, TT, E, C), lambda b, t: (b, t, 0, 0)),
            pl.BlockSpec((1, TT, E, C), lambda b, t: (b, t, 0, 0)),
            pl.BlockSpec((1, 1), lambda b, t: (0, 0)),
        ],
        out_shape=[
            jax.ShapeDtypeStruct((B, T, E, C), jnp.float32),
            jax.ShapeDtypeStruct((B, T, E, C), jnp.float32),
            jax.ShapeDtypeStruct((1, 1), jnp.float32),
        ],
        scratch_shapes=[
            pltpu.VMEM((1, E), jnp.float32),
            pltpu.VMEM((1, E), jnp.float32),
            pltpu.VMEM((1, E), jnp.float32),
            pltpu.VMEM((1, 1), jnp.float32),
        ],
    )(x, w_gating)
    return disp, comb, aux[0, 0]


# speedup vs baseline: 1.6000x; 1.0173x over previous
import functools
import math

import jax
import jax.numpy as jnp
from jax.experimental import pallas as pl
from jax.experimental.pallas import tpu as pltpu

B, T, D, E = 2, 2048, 1024, 8
THRESHOLD = 0.8
CAP_FACTOR_EVAL = 2.0
MIN_EXPERT_CAPACITY = 4
C = max(MIN_EXPERT_CAPACITY, min(T, math.ceil(T * CAP_FACTOR_EVAL / E)))

TT = 512


def _gating_kernel(x_ref, w_ref, disp_ref, comb_ref, aux_ref,
                   carry_ref, accm_ref, accp_ref, auxacc_ref):
    b = pl.program_id(0)
    t = pl.program_id(1)
    n_t = pl.num_programs(1)

    @pl.when(t == 0)
    def _():
        carry_ref[...] = jnp.zeros_like(carry_ref)
        accm_ref[...] = jnp.zeros_like(accm_ref)
        accp_ref[...] = jnp.zeros_like(accp_ref)

    @pl.when((b == 0) & (t == 0))
    def _():
        auxacc_ref[...] = jnp.zeros_like(auxacc_ref)

    xb = x_ref[0]
    gates = jnp.dot(xb, w_ref[...], preferred_element_type=jnp.float32)
    m = jnp.max(gates, axis=-1, keepdims=True)
    ex = jnp.exp(gates - m)
    probs = ex / jnp.sum(ex, axis=-1, keepdims=True)

    ge = probs[:, :, None] <= probs[:, None, :]
    s_incl = jnp.sum(jnp.where(ge, probs[:, None, :], 0.0), axis=-1)
    pmax = jnp.max(probs, axis=-1, keepdims=True)
    sel = (s_incl < THRESHOLD) | (probs >= pmax)
    maskf = sel.astype(jnp.float32)
    selp = probs * maskf
    renorm = jnp.maximum(jnp.sum(selp, axis=-1, keepdims=True), 1e-9)
    weights = selp / renorm

    row = jax.lax.broadcasted_iota(jnp.int32, (TT, TT), 0)
    col = jax.lax.broadcasted_iota(jnp.int32, (TT, TT), 1)
    ltri = (col < row).astype(jnp.float32)
    pos = jnp.dot(ltri, maskf, preferred_element_type=jnp.float32)
    pos = pos + carry_ref[0][None, :]
    carry_ref[0] = carry_ref[0] + jnp.sum(maskf, axis=0)

    accm_ref[0] = accm_ref[0] + jnp.sum(maskf, axis=0)
    accp_ref[0] = accp_ref[0] + jnp.sum(probs, axis=0)

    kc = (pos < float(C)) & sel
    posx = jnp.where(kc, pos, float(C)).astype(jnp.int32)
    ciota = jax.lax.broadcasted_iota(jnp.int32, (TT, E, C), 2)
    onehot = (ciota == posx[:, :, None]).astype(jnp.float32)
    disp_ref[0] = onehot
    comb_ref[0] = onehot * weights[:, :, None]

    @pl.when(t == n_t - 1)
    def _():
        auxacc_ref[...] = auxacc_ref[...] + jnp.sum(
            accm_ref[...] * accp_ref[...], axis=(0, 1), keepdims=True)

    @pl.when((b == B - 1) & (t == n_t - 1))
    def _():
        aux_ref[...] = auxacc_ref[...] * (float(E) / (float(B) * float(T) * float(T)))


@jax.jit
def kernel(x, w_gating):
    n_t = T // TT
    disp, comb, aux = pl.pallas_call(
        _gating_kernel,
        grid=(B, n_t),
        in_specs=[
            pl.BlockSpec((1, TT, D), lambda b, t: (b, t, 0)),
            pl.BlockSpec((D, E), lambda b, t: (0, 0)),
        ],
        out_specs=[
            pl.BlockSpec((1, TT, E, C), lambda b, t: (b, t, 0, 0)),
            pl.BlockSpec((1, TT, E, C), lambda b, t: (b, t, 0, 0)),
            pl.BlockSpec((1, 1), lambda b, t: (0, 0)),
        ],
        out_shape=[
            jax.ShapeDtypeStruct((B, T, E, C), jnp.float32),
            jax.ShapeDtypeStruct((B, T, E, C), jnp.float32),
            jax.ShapeDtypeStruct((1, 1), jnp.float32),
        ],
        scratch_shapes=[
            pltpu.VMEM((1, E), jnp.float32),
            pltpu.VMEM((1, E), jnp.float32),
            pltpu.VMEM((1, E), jnp.float32),
            pltpu.VMEM((1, 1), jnp.float32),
        ],
    )(x, w_gating)
    return disp, comb, aux[0, 0]


# speedup vs baseline: 1.7982x; 1.1238x over previous
import functools
import math

import jax
import jax.numpy as jnp
from jax.experimental import pallas as pl
from jax.experimental.pallas import tpu as pltpu

B, T, D, E = 2, 2048, 1024, 8
THRESHOLD = 0.8
CAP_FACTOR_EVAL = 2.0
MIN_EXPERT_CAPACITY = 4
C = max(MIN_EXPERT_CAPACITY, min(T, math.ceil(T * CAP_FACTOR_EVAL / E)))

TT = 512


def _gating_kernel(x_ref, w_ref, disp_ref, comb_ref, aux_ref,
                   carry_ref, accm_ref, accp_ref, auxacc_ref):
    b = pl.program_id(0)
    t = pl.program_id(1)
    n_t = pl.num_programs(1)

    @pl.when(t == 0)
    def _():
        carry_ref[...] = jnp.zeros_like(carry_ref)
        accm_ref[...] = jnp.zeros_like(accm_ref)
        accp_ref[...] = jnp.zeros_like(accp_ref)

    @pl.when((b == 0) & (t == 0))
    def _():
        auxacc_ref[...] = jnp.zeros_like(auxacc_ref)

    xb = x_ref[0]
    gates = jnp.dot(xb, w_ref[...], preferred_element_type=jnp.float32)
    m = jnp.max(gates, axis=-1, keepdims=True)
    ex = jnp.exp(gates - m)
    probs = ex / jnp.sum(ex, axis=-1, keepdims=True)

    e_of = jax.lax.broadcasted_iota(jnp.int32, (E, E * E), 1) // E
    j_of = jax.lax.broadcasted_iota(jnp.int32, (E, E * E), 1) % E
    src = jax.lax.broadcasted_iota(jnp.int32, (E, E * E), 0)
    rep = (src == e_of).astype(jnp.float32)
    tile = (src == j_of).astype(jnp.float32)
    fold = (jax.lax.broadcasted_iota(jnp.int32, (E * E, E), 0) // E ==
            jax.lax.broadcasted_iota(jnp.int32, (E * E, E), 1)
            ).astype(jnp.float32)
    p_e = jnp.dot(probs, rep, preferred_element_type=jnp.float32)
    p_j = jnp.dot(probs, tile, preferred_element_type=jnp.float32)
    contrib = jnp.where(p_j >= p_e, p_j, 0.0)
    s_incl = jnp.dot(contrib, fold, preferred_element_type=jnp.float32)
    pmax = jnp.max(probs, axis=-1, keepdims=True)
    sel = (s_incl < THRESHOLD) | (probs >= pmax)
    maskf = sel.astype(jnp.float32)
    selp = probs * maskf
    renorm = jnp.maximum(jnp.sum(selp, axis=-1, keepdims=True), 1e-9)
    weights = selp / renorm

    row = jax.lax.broadcasted_iota(jnp.int32, (TT, TT), 0)
    col = jax.lax.broadcasted_iota(jnp.int32, (TT, TT), 1)
    ltri = (col < row).astype(jnp.float32)
    pos = jnp.dot(ltri, maskf, preferred_element_type=jnp.float32)
    pos = pos + carry_ref[0][None, :]
    carry_ref[0] = carry_ref[0] + jnp.sum(maskf, axis=0)

    accm_ref[0] = accm_ref[0] + jnp.sum(maskf, axis=0)
    accp_ref[0] = accp_ref[0] + jnp.sum(probs, axis=0)

    kc = (pos < float(C)) & sel
    posx = jnp.where(kc, pos, float(C)).astype(jnp.int32)
    ciota = jax.lax.broadcasted_iota(jnp.int32, (TT, E, C), 2)
    onehot = (ciota == posx[:, :, None]).astype(jnp.float32)
    disp_ref[0] = onehot
    comb_ref[0] = onehot * weights[:, :, None]

    @pl.when(t == n_t - 1)
    def _():
        auxacc_ref[...] = auxacc_ref[...] + jnp.sum(
            accm_ref[...] * accp_ref[...], axis=(0, 1), keepdims=True)

    @pl.when((b == B - 1) & (t == n_t - 1))
    def _():
        aux_ref[...] = auxacc_ref[...] * (float(E) / (float(B) * float(T) * float(T)))


@jax.jit
def kernel(x, w_gating):
    n_t = T // TT
    disp, comb, aux = pl.pallas_call(
        _gating_kernel,
        grid=(B, n_t),
        in_specs=[
            pl.BlockSpec((1, TT, D), lambda b, t: (b, t, 0)),
            pl.BlockSpec((D, E), lambda b, t: (0, 0)),
        ],
        out_specs=[
            pl.BlockSpec((1, TT, E, C), lambda b, t: (b, t, 0, 0)),
            pl.BlockSpec((1, TT, E, C), lambda b, t: (b, t, 0, 0)),
            pl.BlockSpec((1, 1), lambda b, t: (0, 0)),
        ],
        out_shape=[
            jax.ShapeDtypeStruct((B, T, E, C), jnp.float32),
            jax.ShapeDtypeStruct((B, T, E, C), jnp.float32),
            jax.ShapeDtypeStruct((1, 1), jnp.float32),
        ],
        scratch_shapes=[
            pltpu.VMEM((1, E), jnp.float32),
            pltpu.VMEM((1, E), jnp.float32),
            pltpu.VMEM((1, E), jnp.float32),
            pltpu.VMEM((1, 1), jnp.float32),
        ],
    )(x, w_gating)
    return disp, comb, aux[0, 0]
